# Initial kernel scaffold; baseline (speedup 1.0000x reference)
import jax
import jax.numpy as jnp
from jax.experimental import pallas as pl


def kernel(tag_indices, tag_embeddings):
    raise NotImplementedError("write your pallas kernel here")


# speedup vs baseline: 2.3244x; 2.3244x over previous
import functools

import jax
import jax.numpy as jnp
from jax import lax
from jax.experimental import pallas as pl
from jax.experimental.pallas import tpu as pltpu
from jax.experimental.pallas import tpu_sc as plsc

_NUM_TAGS = 1000
_TAG_DIM = 32
_BATCH = 16384
_SEQ_LEN = 200
_TOTAL = _BATCH * _SEQ_LEN
_NW = 32
_PER_W = _TOTAL // _NW
_CHUNK = 2560
_N_CHUNKS = _PER_W // _CHUNK
_LANES = 16


def _make_kernel():
    mesh = plsc.VectorSubcoreMesh(core_axis_name="c", subcore_axis_name="s")

    @functools.partial(
        pl.kernel,
        mesh=mesh,
        compiler_params=pltpu.CompilerParams(needs_layout_passes=False),
        out_type=jax.ShapeDtypeStruct((_TOTAL * _TAG_DIM,), jnp.float32),
        scratch_types=[
            pltpu.VMEM((_NUM_TAGS * _TAG_DIM,), jnp.float32),
            pltpu.VMEM((_CHUNK,), jnp.int32),
            pltpu.VMEM((_CHUNK * _TAG_DIM,), jnp.float32),
        ],
    )
    def k(idx_hbm, table_hbm, out_hbm, table_v, idx_v, rows_v):
        wid = lax.axis_index("s") * 2 + lax.axis_index("c")
        base = wid * _PER_W
        pltpu.sync_copy(table_hbm, table_v)

        def chunk_body(c, carry):
            off = base + c * _CHUNK
            pltpu.sync_copy(idx_hbm.at[pl.ds(off, _CHUNK)], idx_v)

            def grp(g, carry2):
                idxv = idx_v[pl.ds(g * _LANES, _LANES)]
                rowbase = idxv * _TAG_DIM
                dst0 = lax.iota(jnp.int32, _LANES) * _TAG_DIM + g * (
                    _LANES * _TAG_DIM
                )
                for j in range(_TAG_DIM):
                    vals = plsc.load_gather(table_v, [rowbase + j])
                    plsc.store_scatter(rows_v, [dst0 + j], vals)
                return carry2

            lax.fori_loop(0, _CHUNK // _LANES, grp, 0)
            pltpu.sync_copy(
                rows_v, out_hbm.at[pl.ds(off * _TAG_DIM, _CHUNK * _TAG_DIM)]
            )
            return carry

        lax.fori_loop(0, _N_CHUNKS, chunk_body, 0)

    return k


_gather_kernel = _make_kernel()


def kernel(tag_indices, tag_embeddings):
    idx = tag_indices.reshape(-1).astype(jnp.int32)
    table = tag_embeddings.reshape(-1)
    out = _gather_kernel(idx, table)
    return out.reshape(_BATCH, _SEQ_LEN, _TAG_DIM)


# speedup vs baseline: 3.0188x; 1.2987x over previous
import functools

import jax
import jax.numpy as jnp
from jax import lax
from jax.experimental import pallas as pl
from jax.experimental.pallas import tpu as pltpu
from jax.experimental.pallas import tpu_sc as plsc

_NUM_TAGS = 1000
_TAG_DIM = 32
_BATCH = 16384
_SEQ_LEN = 200
_TOTAL = _BATCH * _SEQ_LEN
_NW = 32
_PER_W = _TOTAL // _NW
_CHUNK = 1280
_N_CHUNKS = _PER_W // _CHUNK
_LANES = 16


def _make_kernel():
    mesh = plsc.VectorSubcoreMesh(core_axis_name="c", subcore_axis_name="s")

    @functools.partial(
        pl.kernel,
        mesh=mesh,
        compiler_params=pltpu.CompilerParams(needs_layout_passes=False),
        out_type=jax.ShapeDtypeStruct((_TOTAL * _TAG_DIM,), jnp.float32),
        scratch_types=[
            pltpu.VMEM((_NUM_TAGS * _TAG_DIM,), jnp.float32),
            pltpu.VMEM((_CHUNK,), jnp.int32),
            pltpu.VMEM((_CHUNK,), jnp.int32),
            pltpu.VMEM((_CHUNK * _TAG_DIM,), jnp.float32),
            pltpu.VMEM((_CHUNK * _TAG_DIM,), jnp.float32),
            pltpu.SemaphoreType.DMA,
            pltpu.SemaphoreType.DMA,
            pltpu.SemaphoreType.DMA,
            pltpu.SemaphoreType.DMA,
        ],
    )
    def k(idx_hbm, table_hbm, out_hbm, table_v, idx_v0, idx_v1, rows_v0,
          rows_v1, si0, si1, so0, so1):
        wid = lax.axis_index("s") * 2 + lax.axis_index("c")
        base = wid * _PER_W
        pltpu.sync_copy(table_hbm, table_v)

        idx_bufs = (idx_v0, idx_v1)
        rows_bufs = (rows_v0, rows_v1)
        sin = (si0, si1)
        sout = (so0, so1)

        def start_in(c, b):
            pltpu.async_copy(
                idx_hbm.at[pl.ds(base + c * _CHUNK, _CHUNK)],
                idx_bufs[b], sin[b],
            )

        def wait_in(b):
            pltpu.make_async_copy(
                idx_hbm.at[pl.ds(0, _CHUNK)], idx_bufs[b], sin[b]
            ).wait()

        def start_out(c, b):
            pltpu.async_copy(
                rows_bufs[b],
                out_hbm.at[
                    pl.ds((base + c * _CHUNK) * _TAG_DIM, _CHUNK * _TAG_DIM)
                ],
                sout[b],
            )

        def wait_out(b):
            pltpu.make_async_copy(
                rows_bufs[b],
                out_hbm.at[pl.ds(0, _CHUNK * _TAG_DIM)],
                sout[b],
            ).wait()

        def compute(b):
            idx_ref = idx_bufs[b]
            rows_ref = rows_bufs[b]

            def grp(g, carry):
                idxv = idx_ref[pl.ds(g * _LANES, _LANES)]
                rowbase = idxv * _TAG_DIM
                dst0 = lax.iota(jnp.int32, _LANES) * _TAG_DIM + g * (
                    _LANES * _TAG_DIM
                )
                for h in range(0, _TAG_DIM, 16):
                    vals = [
                        plsc.load_gather(table_v, [rowbase + (h + j)])
                        for j in range(16)
                    ]
                    for j in range(16):
                        plsc.store_scatter(rows_ref, [dst0 + (h + j)], vals[j])
                return carry

            lax.fori_loop(0, _CHUNK // _LANES, grp, 0)

        start_in(0, 0)
        start_in(1, 1)
        for b in (0, 1):
            wait_in(b)
            compute(b)
            start_out(b, b)
            start_in(2 + b, b)

        def step(t, carry):
            c0 = 2 * t
            for b in (0, 1):
                wait_in(b)
                wait_out(b)
                compute(b)
                start_out(c0 + b, b)
                start_in(c0 + b + 2, b)
            return carry

        lax.fori_loop(1, _N_CHUNKS // 2 - 1, step, 0)

        for b in (0, 1):
            wait_in(b)
            wait_out(b)
            compute(b)
            start_out(_N_CHUNKS - 2 + b, b)
        for b in (0, 1):
            wait_out(b)

    return k


_gather_kernel = _make_kernel()


def kernel(tag_indices, tag_embeddings):
    idx = tag_indices.reshape(-1).astype(jnp.int32)
    table = tag_embeddings.reshape(-1)
    out = _gather_kernel(idx, table)
    return out.reshape(_BATCH, _SEQ_LEN, _TAG_DIM)


# speedup vs baseline: 5.2584x; 1.7419x over previous
import functools

import jax
import jax.numpy as jnp
from jax import lax
from jax.experimental import pallas as pl
from jax.experimental.pallas import tpu as pltpu
from jax.experimental.pallas import tpu_sc as plsc

_NUM_TAGS = 1000
_TAG_DIM = 32
_BATCH = 16384
_SEQ_LEN = 200
_TOTAL = _BATCH * _SEQ_LEN
_NW = 32
_PER_W = _TOTAL // _NW
_CHUNK = 1280
_N_CHUNKS = _PER_W // _CHUNK
_LANES = 16
_HALF = 16


def _make_kernel():
    mesh = plsc.VectorSubcoreMesh(core_axis_name="c", subcore_axis_name="s")

    @functools.partial(
        pl.kernel,
        mesh=mesh,
        compiler_params=pltpu.CompilerParams(needs_layout_passes=False),
        out_type=jax.ShapeDtypeStruct((_TOTAL * _TAG_DIM,), jnp.float32),
        scratch_types=[
            pltpu.VMEM((_NUM_TAGS * _TAG_DIM,), jnp.float32),
            pltpu.VMEM((_CHUNK,), jnp.int32),
            pltpu.VMEM((_CHUNK,), jnp.int32),
            pltpu.VMEM((_CHUNK * _TAG_DIM,), jnp.float32),
            pltpu.VMEM((_CHUNK * _TAG_DIM,), jnp.float32),
            pltpu.SemaphoreType.DMA,
            pltpu.SemaphoreType.DMA,
            pltpu.SemaphoreType.DMA,
            pltpu.SemaphoreType.DMA,
        ],
    )
    def k(idx_hbm, table_hbm, out_hbm, table_v, idx_v0, idx_v1, rows_v0,
          rows_v1, si0, si1, so0, so1):
        wid = lax.axis_index("s") * 2 + lax.axis_index("c")
        base = wid * _PER_W
        pltpu.sync_copy(table_hbm, table_v)

        idx_bufs = (idx_v0, idx_v1)
        rows_bufs = (rows_v0, rows_v1)
        sin = (si0, si1)
        sout = (so0, so1)

        def start_in(c, b):
            pltpu.async_copy(
                idx_hbm.at[pl.ds(base + c * _CHUNK, _CHUNK)],
                idx_bufs[b], sin[b],
            )

        def wait_in(b):
            pltpu.make_async_copy(
                idx_hbm.at[pl.ds(0, _CHUNK)], idx_bufs[b], sin[b]
            ).wait()

        def start_out(c, b):
            pltpu.async_copy(
                rows_bufs[b],
                out_hbm.at[
                    pl.ds((base + c * _CHUNK) * _TAG_DIM, _CHUNK * _TAG_DIM)
                ],
                sout[b],
            )

        def wait_out(b):
            pltpu.make_async_copy(
                rows_bufs[b],
                out_hbm.at[pl.ds(0, _CHUNK * _TAG_DIM)],
                sout[b],
            ).wait()

        def compute(b):
            idx_ref = idx_bufs[b]
            rows_ref = rows_bufs[b]

            def grp(g, carry):
                idxv = idx_ref[pl.ds(g * _LANES, _LANES)]
                dst_base = g * (_LANES * _TAG_DIM)
                for j in range(_LANES):
                    src = idxv[j] * _TAG_DIM
                    for h in (0, _HALF):
                        rows_ref[
                            pl.ds(dst_base + j * _TAG_DIM + h, _HALF)
                        ] = table_v[pl.ds(src + h, _HALF)]
                return carry

            lax.fori_loop(0, _CHUNK // _LANES, grp, 0)

        start_in(0, 0)
        start_in(1, 1)
        for b in (0, 1):
            wait_in(b)
            compute(b)
            start_out(b, b)
            start_in(2 + b, b)

        def step(t, carry):
            c0 = 2 * t
            for b in (0, 1):
                wait_in(b)
                wait_out(b)
                compute(b)
                start_out(c0 + b, b)
                start_in(c0 + b + 2, b)
            return carry

        lax.fori_loop(1, _N_CHUNKS // 2 - 1, step, 0)

        for b in (0, 1):
            wait_in(b)
            wait_out(b)
            compute(b)
            start_out(_N_CHUNKS - 2 + b, b)
        for b in (0, 1):
            wait_out(b)

    return k


_gather_kernel = _make_kernel()


def kernel(tag_indices, tag_embeddings):
    idx = tag_indices.reshape(-1).astype(jnp.int32)
    table = tag_embeddings.reshape(-1)
    out = _gather_kernel(idx, table)
    return out.reshape(_BATCH, _SEQ_LEN, _TAG_DIM)


# speedup vs baseline: 6.9370x; 1.3192x over previous
import functools

import jax
import jax.numpy as jnp
from jax import lax
from jax.experimental import pallas as pl
from jax.experimental.pallas import tpu as pltpu
from jax.experimental.pallas import tpu_sc as plsc

_NUM_TAGS = 1000
_TAG_DIM = 32
_BATCH = 16384
_SEQ_LEN = 200
_NW = 32
_ROWS_PER_W = _BATCH // _NW
_SUPER = 16
_N_SUPER = _ROWS_PER_W // _SUPER
_LANES = 16
_HALF = 16
_N_GROUPS = 13


def _make_kernel():
    mesh = plsc.VectorSubcoreMesh(core_axis_name="c", subcore_axis_name="s")

    @functools.partial(
        pl.kernel,
        mesh=mesh,
        compiler_params=pltpu.CompilerParams(needs_layout_passes=False),
        out_type=jax.ShapeDtypeStruct((_BATCH, _SEQ_LEN, _TAG_DIM),
                                      jnp.float32),
        scratch_types=[
            pltpu.VMEM((_NUM_TAGS * _TAG_DIM,), jnp.float32),
            pltpu.VMEM((_SUPER, _SEQ_LEN), jnp.int32),
            pltpu.VMEM((_SUPER, _SEQ_LEN), jnp.int32),
            pltpu.VMEM((1, _SEQ_LEN, _TAG_DIM), jnp.float32),
            pltpu.VMEM((1, _SEQ_LEN, _TAG_DIM), jnp.float32),
            pltpu.SemaphoreType.DMA,
            pltpu.SemaphoreType.DMA,
            pltpu.SemaphoreType.DMA,
            pltpu.SemaphoreType.DMA,
        ],
    )
    def k(idx_hbm, table_hbm, out_hbm, table_v, idx_v0, idx_v1, rows_v0,
          rows_v1, si0, si1, so0, so1):
        wid = lax.axis_index("s") * 2 + lax.axis_index("c")
        base = wid * _ROWS_PER_W
        pltpu.sync_copy(table_hbm, table_v)

        idx_bufs = (idx_v0, idx_v1)
        rows_bufs = (rows_v0, rows_v1)
        sin = (si0, si1)
        sout = (so0, so1)

        def start_in(s, q):
            pltpu.async_copy(
                idx_hbm.at[pl.ds(base + s * _SUPER, _SUPER)],
                idx_bufs[q], sin[q],
            )

        def wait_in(q):
            pltpu.make_async_copy(
                idx_hbm.at[pl.ds(0, _SUPER)], idx_bufs[q], sin[q]
            ).wait()

        def start_out(brow, p):
            pltpu.async_copy(
                rows_bufs[p], out_hbm.at[pl.ds(brow, 1)], sout[p]
            )

        def wait_out(p):
            pltpu.make_async_copy(
                rows_bufs[p], out_hbm.at[pl.ds(0, 1)], sout[p]
            ).wait()

        def compute(q, r8, p):
            idx_ref = idx_bufs[q]
            rows_ref = rows_bufs[p]

            def grp(g, carry):
                s0 = jnp.where(g == _N_GROUPS - 1, _SEQ_LEN - _LANES,
                               g * _LANES)
                idxv = idx_ref[r8, pl.ds(s0, _LANES)]
                for j in range(_LANES):
                    src = idxv[j] * _TAG_DIM
                    for h in (0, _HALF):
                        rows_ref[0, s0 + j, pl.ds(h, _HALF)] = table_v[
                            pl.ds(src + h, _HALF)
                        ]
                return carry

            lax.fori_loop(0, _N_GROUPS, grp, 0)

        def super_body(s, q):
            wait_in(q)

            def row_pair(r4, carry):
                for p in (0, 1):
                    r8 = r4 * 2 + p
                    wait_out(p)
                    compute(q, r8, p)
                    start_out(base + s * _SUPER + r8, p)
                return carry

            lax.fori_loop(0, _SUPER // 2, row_pair, 0)

        start_in(0, 0)
        start_in(1, 1)
        start_out(base, 0)
        start_out(base + 1, 1)

        def step(t, carry):
            for q in (0, 1):
                super_body(2 * t + q, q)
                start_in(2 * t + q + 2, q)
            return carry

        lax.fori_loop(0, _N_SUPER // 2 - 1, step, 0)

        for q in (0, 1):
            super_body(_N_SUPER - 2 + q, q)
        for p in (0, 1):
            wait_out(p)

    return k


_gather_kernel = _make_kernel()


def kernel(tag_indices, tag_embeddings):
    idx = tag_indices.astype(jnp.int32)
    table = tag_embeddings.reshape(-1)
    return _gather_kernel(idx, table)


# speedup vs baseline: 6.9473x; 1.0015x over previous
import functools

import jax
import jax.numpy as jnp
from jax import lax
from jax.experimental import pallas as pl
from jax.experimental.pallas import tpu as pltpu
from jax.experimental.pallas import tpu_sc as plsc

_NUM_TAGS = 1000
_TAG_DIM = 32
_BATCH = 16384
_SEQ_LEN = 200
_NW = 32
_ROWS_PER_W = _BATCH // _NW
_SUPER = 16
_N_SUPER = _ROWS_PER_W // _SUPER
_LANES = 16
_HALF = 16
_N_GROUPS = 13


def _make_kernel():
    mesh = plsc.VectorSubcoreMesh(core_axis_name="c", subcore_axis_name="s")

    @functools.partial(
        pl.kernel,
        mesh=mesh,
        compiler_params=pltpu.CompilerParams(
            needs_layout_passes=False, use_tc_tiling_on_sc=True
        ),
        out_type=jax.ShapeDtypeStruct((_BATCH, _SEQ_LEN, _TAG_DIM),
                                      jnp.float32),
        scratch_types=[
            pltpu.VMEM((_NUM_TAGS * _TAG_DIM,), jnp.float32),
            pltpu.VMEM((_SUPER, _SEQ_LEN), jnp.int32),
            pltpu.VMEM((_SUPER, _SEQ_LEN), jnp.int32),
            pltpu.VMEM((1, _SEQ_LEN, _TAG_DIM), jnp.float32),
            pltpu.VMEM((1, _SEQ_LEN, _TAG_DIM), jnp.float32),
            pltpu.SemaphoreType.DMA,
            pltpu.SemaphoreType.DMA,
            pltpu.SemaphoreType.DMA,
            pltpu.SemaphoreType.DMA,
        ],
    )
    def k(idx_hbm, table_hbm, out_hbm, table_v, idx_v0, idx_v1, rows_v0,
          rows_v1, si0, si1, so0, so1):
        wid = lax.axis_index("s") * 2 + lax.axis_index("c")
        base = wid * _ROWS_PER_W
        pltpu.sync_copy(table_hbm, table_v)

        idx_bufs = (idx_v0, idx_v1)
        rows_bufs = (rows_v0, rows_v1)
        sin = (si0, si1)
        sout = (so0, so1)

        def start_in(s, q):
            pltpu.async_copy(
                idx_hbm.at[pl.ds(base + s * _SUPER, _SUPER)],
                idx_bufs[q], sin[q],
            )

        def wait_in(q):
            pltpu.make_async_copy(
                idx_hbm.at[pl.ds(0, _SUPER)], idx_bufs[q], sin[q]
            ).wait()

        def start_out(brow, p):
            pltpu.async_copy(
                rows_bufs[p], out_hbm.at[pl.ds(brow, 1)], sout[p]
            )

        def wait_out(p):
            pltpu.make_async_copy(
                rows_bufs[p], out_hbm.at[pl.ds(0, 1)], sout[p]
            ).wait()

        def compute(q, r8, p):
            idx_ref = idx_bufs[q]
            rows_ref = rows_bufs[p]

            def grp(g, carry):
                s0 = jnp.where(g == _N_GROUPS - 1, _SEQ_LEN - _LANES,
                               g * _LANES)
                idxv = idx_ref[r8, pl.ds(s0, _LANES)]
                for j in range(_LANES):
                    src = idxv[j] * _TAG_DIM
                    for h in (0, _HALF):
                        rows_ref[0, s0 + j, pl.ds(h, _HALF)] = table_v[
                            pl.ds(src + h, _HALF)
                        ]
                return carry

            lax.fori_loop(0, _N_GROUPS, grp, 0)

        def super_body(s, q):
            wait_in(q)

            def row_pair(r4, carry):
                for p in (0, 1):
                    r8 = r4 * 2 + p
                    wait_out(p)
                    compute(q, r8, p)
                    start_out(base + s * _SUPER + r8, p)
                return carry

            lax.fori_loop(0, _SUPER // 2, row_pair, 0)

        start_in(0, 0)
        start_in(1, 1)
        start_out(base, 0)
        start_out(base + 1, 1)

        def step(t, carry):
            for q in (0, 1):
                super_body(2 * t + q, q)
                start_in(2 * t + q + 2, q)
            return carry

        lax.fori_loop(0, _N_SUPER // 2 - 1, step, 0)

        for q in (0, 1):
            super_body(_N_SUPER - 2 + q, q)
        for p in (0, 1):
            wait_out(p)

    return k


_gather_kernel = _make_kernel()


def kernel(tag_indices, tag_embeddings):
    idx = tag_indices.astype(jnp.int32)
    table = tag_embeddings.reshape(-1)
    return _gather_kernel(idx, table)


# speedup vs baseline: 14.0449x; 2.0216x over previous
import functools

import jax
import jax.numpy as jnp
from jax import lax
from jax.experimental import pallas as pl
from jax.experimental.pallas import tpu as pltpu
from jax.experimental.pallas import tpu_sc as plsc

_NUM_TAGS = 1000
_TAG_DIM = 32
_BATCH = 16384
_SEQ_LEN = 200
_NW = 32
_BW = _BATCH // _NW
_SBLK = 8
_N_SBLK = _SEQ_LEN // _SBLK
_LANES = 16
_NGRP = _BW // _LANES


def _make_kernel():
    mesh = plsc.VectorSubcoreMesh(core_axis_name="c", subcore_axis_name="s")

    @functools.partial(
        pl.kernel,
        mesh=mesh,
        compiler_params=pltpu.CompilerParams(
            needs_layout_passes=False, use_tc_tiling_on_sc=True
        ),
        out_type=jax.ShapeDtypeStruct((_SEQ_LEN, _TAG_DIM, _BATCH),
                                      jnp.float32),
        scratch_types=[
            pltpu.VMEM((_TAG_DIM * _NUM_TAGS,), jnp.float32),
            pltpu.VMEM((_SBLK, _BW), jnp.int32),
            pltpu.VMEM((_SBLK, _BW), jnp.int32),
            pltpu.VMEM((1, _TAG_DIM, _BW), jnp.float32),
            pltpu.VMEM((1, _TAG_DIM, _BW), jnp.float32),
            pltpu.SemaphoreType.DMA,
            pltpu.SemaphoreType.DMA,
            pltpu.SemaphoreType.DMA,
            pltpu.SemaphoreType.DMA,
        ],
    )
    def k(idxt_hbm, tabt_hbm, out_hbm, tab_v, idx_v0, idx_v1, row_v0,
          row_v1, si0, si1, so0, so1):
        wid = lax.axis_index("s") * 2 + lax.axis_index("c")
        b0 = wid * _BW
        pltpu.sync_copy(tabt_hbm, tab_v)

        idx_bufs = (idx_v0, idx_v1)
        row_bufs = (row_v0, row_v1)
        sin = (si0, si1)
        sout = (so0, so1)

        def start_in(blk, q):
            pltpu.async_copy(
                idxt_hbm.at[pl.ds(blk * _SBLK, _SBLK), pl.ds(b0, _BW)],
                idx_bufs[q], sin[q],
            )

        def wait_in(q):
            pltpu.make_async_copy(
                idxt_hbm.at[pl.ds(0, _SBLK), pl.ds(0, _BW)],
                idx_bufs[q], sin[q],
            ).wait()

        def start_out(s, p):
            pltpu.async_copy(
                row_bufs[p],
                out_hbm.at[pl.ds(s, 1), :, pl.ds(b0, _BW)],
                sout[p],
            )

        def wait_out(p):
            pltpu.make_async_copy(
                row_bufs[p],
                out_hbm.at[pl.ds(0, 1), :, pl.ds(0, _BW)],
                sout[p],
            ).wait()

        def compute(q, sl, p):
            idx_ref = idx_bufs[q]
            row_ref = row_bufs[p]

            def grp(bg, carry):
                idxv = idx_ref[sl, pl.ds(bg * _LANES, _LANES)]
                for d in range(_TAG_DIM):
                    vals = plsc.load_gather(tab_v, [idxv + d * _NUM_TAGS])
                    row_ref[0, d, pl.ds(bg * _LANES, _LANES)] = vals
                return carry

            lax.fori_loop(0, _NGRP, grp, 0)

        def sblock(blk, q):
            wait_in(q)

            def spair(s2, carry):
                for p in (0, 1):
                    sl = s2 * 2 + p
                    wait_out(p)
                    compute(q, sl, p)
                    start_out(blk * _SBLK + sl, p)
                return carry

            lax.fori_loop(0, _SBLK // 2, spair, 0)

        start_in(0, 0)
        start_in(1, 1)
        start_out(0, 0)
        start_out(1, 1)

        sblock(0, 0)
        start_in(2, 0)

        def step(t, carry):
            for q in (1, 0):
                blk = 2 * t + 1 + (1 - q)
                sblock(blk, q)
                start_in(blk + 2, q)
            return carry

        lax.fori_loop(0, _N_SBLK // 2 - 1, step, 0)

        sblock(_N_SBLK - 2, 1)
        sblock(_N_SBLK - 1, 0)
        for p in (0, 1):
            wait_out(p)

    return k


_gather_kernel = _make_kernel()


def kernel(tag_indices, tag_embeddings):
    idxt = tag_indices.T.astype(jnp.int32)
    tabt = tag_embeddings.T.reshape(-1)
    out = _gather_kernel(idxt, tabt)
    return jnp.transpose(out, (2, 0, 1))


# speedup vs baseline: 40.9691x; 2.9170x over previous
import functools

import jax
import jax.numpy as jnp
from jax import lax
from jax.experimental import pallas as pl
from jax.experimental.pallas import tpu as pltpu
from jax.experimental.pallas import tpu_sc as plsc

_NUM_TAGS = 1000
_TAG_DIM = 32
_BATCH = 16384
_SEQ_LEN = 200
_NW = 32
_BW = _BATCH // _NW
_SBLK = 8
_N_SBLK = _SEQ_LEN // _SBLK
_LANES = 16
_NGRP = _BW // _LANES


def _make_kernel():
    mesh = plsc.VectorSubcoreMesh(core_axis_name="c", subcore_axis_name="s")

    @functools.partial(
        pl.kernel,
        mesh=mesh,
        compiler_params=pltpu.CompilerParams(
            needs_layout_passes=False, use_tc_tiling_on_sc=True
        ),
        out_type=jax.ShapeDtypeStruct((_SEQ_LEN, _TAG_DIM, _BATCH),
                                      jnp.float32),
        scratch_types=[
            pltpu.VMEM((_TAG_DIM * _NUM_TAGS,), jnp.float32),
            pltpu.VMEM((_SBLK, _BW), jnp.int32),
            pltpu.VMEM((_SBLK, _BW), jnp.int32),
            pltpu.VMEM((1, _TAG_DIM, _BW), jnp.float32),
            pltpu.VMEM((1, _TAG_DIM, _BW), jnp.float32),
            pltpu.SemaphoreType.DMA,
            pltpu.SemaphoreType.DMA,
            pltpu.SemaphoreType.DMA,
            pltpu.SemaphoreType.DMA,
        ],
    )
    def k(idxt_hbm, tabt_hbm, out_hbm, tab_v, idx_v0, idx_v1, row_v0,
          row_v1, si0, si1, so0, so1):
        wid = lax.axis_index("s") * 2 + lax.axis_index("c")
        b0 = wid * _BW
        pltpu.sync_copy(tabt_hbm, tab_v)

        idx_bufs = (idx_v0, idx_v1)
        row_bufs = (row_v0, row_v1)
        sin = (si0, si1)
        sout = (so0, so1)

        def start_in(blk, q):
            pltpu.async_copy(
                idxt_hbm.at[pl.ds(blk * _SBLK, _SBLK), pl.ds(b0, _BW)],
                idx_bufs[q], sin[q],
            )

        def wait_in(q):
            pltpu.make_async_copy(
                idxt_hbm.at[pl.ds(0, _SBLK), pl.ds(0, _BW)],
                idx_bufs[q], sin[q],
            ).wait()

        def start_out(s, p):
            pltpu.async_copy(
                row_bufs[p],
                out_hbm.at[pl.ds(s, 1), :, pl.ds(b0, _BW)],
                sout[p],
            )

        def wait_out(p):
            pltpu.make_async_copy(
                row_bufs[p],
                out_hbm.at[pl.ds(0, 1), :, pl.ds(0, _BW)],
                sout[p],
            ).wait()

        def compute(q, sl, p):
            idx_ref = idx_bufs[q]
            row_ref = row_bufs[p]

            def grp(bg, carry):
                idxv = idx_ref[sl, pl.ds(bg * _LANES, _LANES)]
                vals = [
                    plsc.load_gather(tab_v, [idxv + d * _NUM_TAGS])
                    for d in range(_TAG_DIM)
                ]
                for d in range(_TAG_DIM):
                    row_ref[0, d, pl.ds(bg * _LANES, _LANES)] = vals[d]
                return carry

            lax.fori_loop(0, _NGRP, grp, 0)

        def sblock(blk, q):
            wait_in(q)

            def spair(s2, carry):
                for p in (0, 1):
                    sl = s2 * 2 + p
                    wait_out(p)
                    compute(q, sl, p)
                    start_out(blk * _SBLK + sl, p)
                return carry

            lax.fori_loop(0, _SBLK // 2, spair, 0)

        start_in(0, 0)
        start_in(1, 1)
        start_out(0, 0)
        start_out(1, 1)

        sblock(0, 0)
        start_in(2, 0)

        def step(t, carry):
            for q in (1, 0):
                blk = 2 * t + 1 + (1 - q)
                sblock(blk, q)
                start_in(blk + 2, q)
            return carry

        lax.fori_loop(0, _N_SBLK // 2 - 1, step, 0)

        sblock(_N_SBLK - 2, 1)
        sblock(_N_SBLK - 1, 0)
        for p in (0, 1):
            wait_out(p)

    return k


_gather_kernel = _make_kernel()


def kernel(tag_indices, tag_embeddings):
    idxt = tag_indices.T.astype(jnp.int32)
    tabt = tag_embeddings.T.reshape(-1)
    out = _gather_kernel(idxt, tabt)
    return jnp.transpose(out, (2, 0, 1))


# speedup vs baseline: 41.1938x; 1.0055x over previous
import functools

import jax
import jax.numpy as jnp
from jax import lax
from jax.experimental import pallas as pl
from jax.experimental.pallas import tpu as pltpu
from jax.experimental.pallas import tpu_sc as plsc

_NUM_TAGS = 1000
_TAG_DIM = 32
_BATCH = 16384
_SEQ_LEN = 200
_NW = 32
_BW = _BATCH // _NW
_SBLK = 8
_N_SBLK = _SEQ_LEN // _SBLK
_LANES = 16
_NGRP = _BW // _LANES


def _make_kernel():
    mesh = plsc.VectorSubcoreMesh(core_axis_name="c", subcore_axis_name="s")

    @functools.partial(
        pl.kernel,
        mesh=mesh,
        compiler_params=pltpu.CompilerParams(
            needs_layout_passes=False, use_tc_tiling_on_sc=True
        ),
        out_type=jax.ShapeDtypeStruct((_SEQ_LEN, _TAG_DIM, _BATCH),
                                      jnp.float32),
        scratch_types=[
            pltpu.VMEM((_TAG_DIM * _NUM_TAGS,), jnp.float32),
            pltpu.VMEM((_SBLK, _BW), jnp.int32),
            pltpu.VMEM((_SBLK, _BW), jnp.int32),
            pltpu.VMEM((1, _TAG_DIM, _BW), jnp.float32),
            pltpu.VMEM((1, _TAG_DIM, _BW), jnp.float32),
            pltpu.SemaphoreType.DMA,
            pltpu.SemaphoreType.DMA,
            pltpu.SemaphoreType.DMA,
            pltpu.SemaphoreType.DMA,
        ],
    )
    def k(idxt_hbm, tabt_hbm, out_hbm, tab_v, idx_v0, idx_v1, row_v0,
          row_v1, si0, si1, so0, so1):
        wid = lax.axis_index("s") * 2 + lax.axis_index("c")
        b0 = wid * _BW
        pltpu.sync_copy(tabt_hbm, tab_v)

        idx_bufs = (idx_v0, idx_v1)
        row_bufs = (row_v0, row_v1)
        sin = (si0, si1)
        sout = (so0, so1)

        def start_in(blk, q):
            pltpu.async_copy(
                idxt_hbm.at[pl.ds(blk * _SBLK, _SBLK), pl.ds(b0, _BW)],
                idx_bufs[q], sin[q],
            )

        def wait_in(q):
            pltpu.make_async_copy(
                idxt_hbm.at[pl.ds(0, _SBLK), pl.ds(0, _BW)],
                idx_bufs[q], sin[q],
            ).wait()

        def start_out(s, p):
            pltpu.async_copy(
                row_bufs[p],
                out_hbm.at[pl.ds(s, 1), :, pl.ds(b0, _BW)],
                sout[p],
            )

        def wait_out(p):
            pltpu.make_async_copy(
                row_bufs[p],
                out_hbm.at[pl.ds(0, 1), :, pl.ds(0, _BW)],
                sout[p],
            ).wait()

        def compute(q, sl, p):
            idx_ref = idx_bufs[q]
            row_ref = row_bufs[p]

            @plsc.parallel_loop(0, _NGRP, unroll=2)
            def grp(bg):
                idxv = idx_ref[sl, pl.ds(bg * _LANES, _LANES)]
                vals = [
                    plsc.load_gather(tab_v, [idxv + d * _NUM_TAGS])
                    for d in range(_TAG_DIM)
                ]
                for d in range(_TAG_DIM):
                    row_ref[0, d, pl.ds(bg * _LANES, _LANES)] = vals[d]

        def sblock(blk, q):
            wait_in(q)

            def spair(s2, carry):
                for p in (0, 1):
                    sl = s2 * 2 + p
                    wait_out(p)
                    compute(q, sl, p)
                    start_out(blk * _SBLK + sl, p)
                return carry

            lax.fori_loop(0, _SBLK // 2, spair, 0)

        start_in(0, 0)
        start_in(1, 1)
        start_out(0, 0)
        start_out(1, 1)

        sblock(0, 0)
        start_in(2, 0)

        def step(t, carry):
            for q in (1, 0):
                blk = 2 * t + 1 + (1 - q)
                sblock(blk, q)
                start_in(blk + 2, q)
            return carry

        lax.fori_loop(0, _N_SBLK // 2 - 1, step, 0)

        sblock(_N_SBLK - 2, 1)
        sblock(_N_SBLK - 1, 0)
        for p in (0, 1):
            wait_out(p)

    return k


_gather_kernel = _make_kernel()


def kernel(tag_indices, tag_embeddings):
    idxt = tag_indices.T.astype(jnp.int32)
    tabt = tag_embeddings.T.reshape(-1)
    out = _gather_kernel(idxt, tabt)
    return jnp.transpose(out, (2, 0, 1))


# speedup vs baseline: 63.4701x; 1.5408x over previous
import functools

import jax
import jax.numpy as jnp
from jax import lax
from jax.experimental import pallas as pl
from jax.experimental.pallas import tpu as pltpu
from jax.experimental.pallas import tpu_sc as plsc

_NUM_TAGS = 1000
_TAG_DIM = 32
_BATCH = 16384
_SEQ_LEN = 200
_NW = 32
_BW = _BATCH // _NW
_SBLK = 8
_N_SBLK = _SEQ_LEN // _SBLK
_LANES = 16
_NGRP = _BW // _LANES
_NOUT = 4


def _make_kernel():
    mesh = plsc.VectorSubcoreMesh(core_axis_name="c", subcore_axis_name="s")

    @functools.partial(
        pl.kernel,
        mesh=mesh,
        compiler_params=pltpu.CompilerParams(
            needs_layout_passes=False, use_tc_tiling_on_sc=True
        ),
        out_type=jax.ShapeDtypeStruct((_SEQ_LEN, _TAG_DIM, _BATCH),
                                      jnp.float32),
        scratch_types=[
            pltpu.VMEM((_TAG_DIM * _NUM_TAGS,), jnp.float32),
            pltpu.VMEM((_SBLK, _BW), jnp.int32),
            pltpu.VMEM((_SBLK, _BW), jnp.int32),
            pltpu.VMEM((1, _TAG_DIM, _BW), jnp.float32),
            pltpu.VMEM((1, _TAG_DIM, _BW), jnp.float32),
            pltpu.VMEM((1, _TAG_DIM, _BW), jnp.float32),
            pltpu.VMEM((1, _TAG_DIM, _BW), jnp.float32),
            pltpu.SemaphoreType.DMA,
            pltpu.SemaphoreType.DMA,
            pltpu.SemaphoreType.DMA,
            pltpu.SemaphoreType.DMA,
            pltpu.SemaphoreType.DMA,
            pltpu.SemaphoreType.DMA,
        ],
    )
    def k(idxt_hbm, tabt_hbm, out_hbm, tab_v, idx_v0, idx_v1, row_v0,
          row_v1, row_v2, row_v3, si0, si1, so0, so1, so2, so3):
        wid = lax.axis_index("s") * 2 + lax.axis_index("c")
        b0 = wid * _BW
        pltpu.sync_copy(tabt_hbm, tab_v)

        idx_bufs = (idx_v0, idx_v1)
        row_bufs = (row_v0, row_v1, row_v2, row_v3)
        sin = (si0, si1)
        sout = (so0, so1, so2, so3)

        def start_in(blk, q):
            pltpu.async_copy(
                idxt_hbm.at[pl.ds(blk * _SBLK, _SBLK), pl.ds(b0, _BW)],
                idx_bufs[q], sin[q],
            )

        def wait_in(q):
            pltpu.make_async_copy(
                idxt_hbm.at[pl.ds(0, _SBLK), pl.ds(0, _BW)],
                idx_bufs[q], sin[q],
            ).wait()

        def start_out(s, p):
            pltpu.async_copy(
                row_bufs[p],
                out_hbm.at[pl.ds(s, 1), :, pl.ds(b0, _BW)],
                sout[p],
            )

        def wait_out(p):
            pltpu.make_async_copy(
                row_bufs[p],
                out_hbm.at[pl.ds(0, 1), :, pl.ds(0, _BW)],
                sout[p],
            ).wait()

        def compute(q, sl, p):
            idx_ref = idx_bufs[q]
            row_ref = row_bufs[p]

            @plsc.parallel_loop(0, _NGRP)
            def grp(bg):
                idxv = idx_ref[sl, pl.ds(bg * _LANES, _LANES)]
                vals = [
                    plsc.load_gather(tab_v, [idxv + d * _NUM_TAGS])
                    for d in range(_TAG_DIM)
                ]
                for d in range(_TAG_DIM):
                    row_ref[0, d, pl.ds(bg * _LANES, _LANES)] = vals[d]

        def sblock(blk, q):
            wait_in(q)

            def squad(s4, carry):
                for p in range(_NOUT):
                    sl = s4 * _NOUT + p
                    wait_out(p)
                    compute(q, sl, p)
                    start_out(blk * _SBLK + sl, p)
                return carry

            lax.fori_loop(0, _SBLK // _NOUT, squad, 0)

        start_in(0, 0)
        start_in(1, 1)
        for p in range(_NOUT):
            start_out(p, p)

        sblock(0, 0)
        start_in(2, 0)

        def step(t, carry):
            for q in (1, 0):
                blk = 2 * t + 1 + (1 - q)
                sblock(blk, q)
                start_in(blk + 2, q)
            return carry

        lax.fori_loop(0, _N_SBLK // 2 - 1, step, 0)

        sblock(_N_SBLK - 2, 1)
        sblock(_N_SBLK - 1, 0)
        for p in range(_NOUT):
            wait_out(p)

    return k


_gather_kernel = _make_kernel()


def kernel(tag_indices, tag_embeddings):
    idxt = tag_indices.T.astype(jnp.int32)
    tabt = tag_embeddings.T.reshape(-1)
    out = _gather_kernel(idxt, tabt)
    return jnp.transpose(out, (2, 0, 1))
